# baseline (speedup 1.0000x reference)
import jax
import jax.numpy as jnp
from jax.experimental import pallas as pl
from jax.experimental.pallas import tpu as pltpu

_BANK = 100000
_BATCH = 16384
_D = 128
_TAIL = _BANK - _BATCH


_NSPLIT = 16
_UNITS = _TAIL // 8
_SPLIT_ROWS = [
    8 * (_UNITS // _NSPLIT + (1 if i < _UNITS % _NSPLIT else 0))
    for i in range(_NSPLIT)
]


def _body(feats_hbm, bank_hbm, out_hbm, x_vmem, y_vmem, sem_tail, sem_in, sem_out):
    tails = []
    off = _BATCH
    for i in range(_NSPLIT):
        rows = _SPLIT_ROWS[i]
        t = pltpu.make_async_copy(
            bank_hbm.at[pl.ds(off, rows)],
            out_hbm.at[pl.ds(off, rows)],
            sem_tail.at[i],
        )
        t.start()
        tails.append(t)
        off += rows
    feats_in = pltpu.make_async_copy(feats_hbm, x_vmem, sem_in)
    feats_in.start()
    feats_in.wait()
    x = x_vmem[...]
    n2 = jnp.sum(x * x, axis=1, keepdims=True)
    y_vmem[...] = x * jax.lax.rsqrt(jnp.maximum(n2, 1e-24))
    head = pltpu.make_async_copy(y_vmem, out_hbm.at[pl.ds(0, _BATCH)], sem_out)
    head.start()
    head.wait()
    for t in tails:
        t.wait()


def kernel(feats, bank):
    return pl.pallas_call(
        _body,
        in_specs=[
            pl.BlockSpec(memory_space=pltpu.MemorySpace.HBM),
            pl.BlockSpec(memory_space=pltpu.MemorySpace.HBM),
        ],
        out_specs=pl.BlockSpec(memory_space=pltpu.MemorySpace.HBM),
        out_shape=jax.ShapeDtypeStruct((_BANK, _D), jnp.float32),
        scratch_shapes=[
            pltpu.VMEM((_BATCH, _D), jnp.float32),
            pltpu.VMEM((_BATCH, _D), jnp.float32),
            pltpu.SemaphoreType.DMA((_NSPLIT,)),
            pltpu.SemaphoreType.DMA,
            pltpu.SemaphoreType.DMA,
        ],
    )(feats, bank)


# speedup vs baseline: 38.5433x; 38.5433x over previous
import jax
import jax.numpy as jnp
from jax.experimental import pallas as pl

_BANK = 100000
_BATCH = 16384
_D = 128
_BLK = 8192
_NFEAT_BLKS = _BATCH // _BLK
_NBLKS = (_BANK + _BLK - 1) // _BLK


def _body(feats_ref, bank_ref, out_ref):
    i = pl.program_id(0)

    @pl.when(i < _NFEAT_BLKS)
    def _():
        x = feats_ref[...]
        n2 = jnp.sum(x * x, axis=1, keepdims=True)
        out_ref[...] = x * jax.lax.rsqrt(jnp.maximum(n2, 1e-24))

    @pl.when(i >= _NFEAT_BLKS)
    def _():
        out_ref[...] = bank_ref[...]


def kernel(feats, bank):
    return pl.pallas_call(
        _body,
        grid=(_NBLKS,),
        in_specs=[
            pl.BlockSpec((_BLK, _D), lambda i: (jnp.minimum(i, _NFEAT_BLKS - 1), 0)),
            pl.BlockSpec((_BLK, _D), lambda i: (jnp.maximum(i, _NFEAT_BLKS), 0)),
        ],
        out_specs=pl.BlockSpec((_BLK, _D), lambda i: (i, 0)),
        out_shape=jax.ShapeDtypeStruct((_BANK, _D), jnp.float32),
    )(feats, bank)
